# baseline (speedup 1.0000x reference)
import functools

import jax
import jax.numpy as jnp
from jax import lax
from jax.experimental import pallas as pl
from jax.experimental.pallas import tpu as pltpu
from jax.experimental.pallas import tpu_sc as plsc

B = 16384
CONT = 13
NF = 26
V = 100000
D = 16

NC = 2
NS = 16
NW = NC * NS
ROWS_W = B // NW
G = 128
NG_W = ROWS_W * NF // G
CB = 64
CHUNK_G = CB * NF // G
NCHUNK = ROWS_W // CB
PERIOD = 13
OUT_W = CONT + NF * D


def _sc_kernel(cate_hbm, cont_hbm, offs_hbm, table_hbm, out_hbm,
               idx_v, off_v, em_v, cont_v, row_v, gsem):
    wid = lax.axis_index("s") * NC + lax.axis_index("c")

    pltpu.sync_copy(cate_hbm.at[pl.ds(wid * NG_W, NG_W)], idx_v)
    pltpu.sync_copy(offs_hbm, off_v)

    def fix_body(g, carry):
        p = lax.rem(g, PERIOD)
        for k in range(G // 16):
            s = pl.ds(k * 16, 16)
            idx_v[g, s] = idx_v[g, s] + off_v[p, s]
        return carry

    lax.fori_loop(0, NG_W, fix_body, 0)

    def fire(c, buf):
        for j in range(CHUNK_G):
            pltpu.async_copy(
                table_hbm.at[idx_v.at[c * CHUNK_G + j]],
                em_v.at[buf].at[pl.ds(j * G, G)], gsem)

    def drain(c, buf):
        for j in range(CHUNK_G):
            pltpu.make_async_copy(
                table_hbm.at[idx_v.at[c * CHUNK_G + j]],
                em_v.at[buf].at[pl.ds(j * G, G)], gsem).wait()

    def assemble(b, buf):
        row_v[b, pl.ds(0, 16)] = cont_v[b]
        for f in range(NF):
            row_v[b, pl.ds(CONT + f * D, D)] = em_v[buf, NF * b + f]
        return buf

    fire(0, 0)

    def chunk_body(c, carry):
        buf = lax.rem(c, 2)
        row0 = wid * ROWS_W + c * CB
        pltpu.sync_copy(cont_hbm.at[pl.ds(row0, CB)], cont_v)
        drain(c, buf)
        nc = c + 1

        @pl.when(nc < NCHUNK)
        def _():
            fire(nc, lax.rem(nc, 2))

        lax.fori_loop(0, CB, assemble, buf)
        pltpu.sync_copy(row_v, out_hbm.at[pl.ds(row0, CB)])
        return carry

    lax.fori_loop(0, NCHUNK, chunk_body, 0)


@jax.jit
def kernel(x_cont, x_cate, tables):
    cate2d = x_cate.reshape(B * NF // G, G)
    cont_pad = jnp.pad(x_cont, ((0, 0), (0, 3)))
    table_flat = tables.reshape(NF * V, D)
    offs = ((jnp.arange(PERIOD * G, dtype=jnp.int32) % NF) * V
            ).reshape(PERIOD, G)
    mesh = plsc.VectorSubcoreMesh(core_axis_name="c", subcore_axis_name="s")
    run = functools.partial(
        pl.kernel,
        mesh=mesh,
        compiler_params=pltpu.CompilerParams(use_tc_tiling_on_sc=False),
        out_type=jax.ShapeDtypeStruct((B, OUT_W), jnp.float32),
        scratch_types=[
            pltpu.VMEM((NG_W, G), jnp.int32),
            pltpu.VMEM((PERIOD, G), jnp.int32),
            pltpu.VMEM((2, CHUNK_G * G, D), jnp.float32),
            pltpu.VMEM((CB, 16), jnp.float32),
            pltpu.VMEM((CB, OUT_W), jnp.float32),
            pltpu.SemaphoreType.DMA,
        ],
    )(_sc_kernel)
    return run(cate2d, cont_pad, offs, table_flat)
